# baseline (speedup 1.0000x reference)
import functools

import jax
import jax.numpy as jnp
from jax import lax
from jax.experimental import pallas as pl
from jax.experimental.pallas import tpu as pltpu
from jax.experimental.pallas import tpu_sc as plsc

N = 100000
K = 20
D = 32
CHUNK = 32
E = CHUNK * K
NCHUNKS = N // CHUNK
NW = 32
GCALL = 640
NGC = E // GCALL
MAXT = (NCHUNKS + NW - 1) // NW
HALFT = MAXT // 2


def _body(nei_hbm, wei_hbm, emb_hbm, out_hbm,
          idx_v, wvec_v, rows_v, out_v,
          sem_i0, sem_i1, sem_g0, sem_g1, sem_w0, sem_w1, sem_o0, sem_o1):
    cid = lax.axis_index("c")
    sid = lax.axis_index("s")
    wid = sid * 2 + cid
    sem_i = (sem_i0, sem_i1)
    sem_g = (sem_g0, sem_g1)
    sem_w = (sem_w0, sem_w1)
    sem_o = (sem_o0, sem_o1)

    def chunk_of(t):
        return wid + t * NW

    def start_idx(t, b):
        @pl.when(chunk_of(t) < NCHUNKS)
        def _():
            pltpu.async_copy(nei_hbm.at[pl.ds(chunk_of(t) * E, E)],
                             idx_v.at[b], sem_i[b])

    def wait_idx(t, b):
        @pl.when(chunk_of(t) < NCHUNKS)
        def _():
            pltpu.make_async_copy(nei_hbm.at[pl.ds(chunk_of(t) * E, E)],
                                  idx_v.at[b], sem_i[b]).wait()

    def start_gather(t, b):
        @pl.when(chunk_of(t) < NCHUNKS)
        def _():
            for j in range(NGC):
                pltpu.async_copy(
                    emb_hbm.at[idx_v.at[b, pl.ds(j * GCALL, GCALL)]],
                    rows_v.at[b, pl.ds(j * GCALL, GCALL)],
                    sem_g[b],
                )
            pltpu.async_copy(
                wei_hbm.at[pl.ds(chunk_of(t) * CHUNK * 32, CHUNK * 32)],
                wvec_v.at[b], sem_w[b])

    def wait_gather(t, b):
        @pl.when(chunk_of(t) < NCHUNKS)
        def _():
            for j in range(NGC):
                pltpu.make_async_copy(
                    emb_hbm.at[idx_v.at[b, pl.ds(j * GCALL, GCALL)]],
                    rows_v.at[b, pl.ds(j * GCALL, GCALL)],
                    sem_g[b],
                ).wait()
            pltpu.make_async_copy(
                wei_hbm.at[pl.ds(chunk_of(t) * CHUNK * 32, CHUNK * 32)],
                wvec_v.at[b], sem_w[b])\
                .wait()

    def wait_out(t, b):
        c = chunk_of(t)

        @pl.when(jnp.logical_and(t >= 0, c < NCHUNKS))
        def _():
            pltpu.make_async_copy(
                out_v.at[b], out_hbm.at[pl.ds(chunk_of(t) * CHUNK, CHUNK)],
                sem_o[b]).wait()

    def compute(t, b):
        @pl.when(chunk_of(t) < NCHUNKS)
        def _():
            def row_body(r, carry2):
                e0 = r * K
                wa = wvec_v[b, pl.ds(r * 32, 16)]
                wb = wvec_v[b, pl.ds(r * 32 + 16, 16)]
                w0 = wa[0]
                acc0 = w0 * rows_v[b, e0, pl.ds(0, 16)]
                acc1 = w0 * rows_v[b, e0, pl.ds(16, 16)]
                for k in range(1, K):
                    w = wa[k] if k < 16 else wb[k - 16]
                    acc0 = acc0 + w * rows_v[b, e0 + k, pl.ds(0, 16)]
                    acc1 = acc1 + w * rows_v[b, e0 + k, pl.ds(16, 16)]
                out_v[b, r, 0, pl.ds(0, 16)] = acc0
                out_v[b, r, 0, pl.ds(16, 16)] = acc1
                return carry2

            lax.fori_loop(0, CHUNK, row_body, 0, unroll=4)
            pltpu.async_copy(
                out_v.at[b], out_hbm.at[pl.ds(chunk_of(t) * CHUNK, CHUNK)],
                sem_o[b])

    start_idx(0, 0)
    start_idx(1, 1)
    wait_idx(0, 0)
    start_gather(0, 0)

    def step(u, carry):
        t0 = 2 * u
        wait_idx(t0 + 1, 1)
        start_gather(t0 + 1, 1)
        wait_gather(t0, 0)
        start_idx(t0 + 2, 0)
        wait_out(t0 - 2, 0)
        compute(t0, 0)
        wait_idx(t0 + 2, 0)
        start_gather(t0 + 2, 0)
        wait_gather(t0 + 1, 1)
        start_idx(t0 + 3, 1)
        wait_out(t0 - 1, 1)
        compute(t0 + 1, 1)
        return carry

    lax.fori_loop(0, HALFT, step, 0)
    wait_out(MAXT - 2, 0)
    wait_out(MAXT - 1, 1)


@jax.jit
def _pprgo(nei_flat, wei_flat, emb_table):
    mesh = plsc.VectorSubcoreMesh(core_axis_name="c", subcore_axis_name="s")
    out = pl.kernel(
        _body,
        out_type=jax.ShapeDtypeStruct((N, 1, D), jnp.float32),
        mesh=mesh,
        scratch_types=[
            pltpu.VMEM((2, E), jnp.int32),
            pltpu.VMEM((2, CHUNK * 32), jnp.float32),
            pltpu.VMEM((2, E, D), jnp.float32),
            pltpu.VMEM((2, CHUNK, 1, D), jnp.float32),
        ] + [pltpu.SemaphoreType.DMA] * 8,
        compiler_params=pltpu.CompilerParams(use_tc_tiling_on_sc=False),
    )(nei_flat, wei_flat, emb_table)
    return out


def kernel(nei, wei, emb_table):
    nei_flat = nei.reshape(-1).astype(jnp.int32)
    wei_pad = jnp.pad(wei, ((0, 0), (0, 32 - K))).reshape(-1)
    return _pprgo(nei_flat, wei_pad, emb_table)


# speedup vs baseline: 1.3277x; 1.3277x over previous
import functools

import jax
import jax.numpy as jnp
from jax import lax
from jax.experimental import pallas as pl
from jax.experimental.pallas import tpu as pltpu
from jax.experimental.pallas import tpu_sc as plsc

N = 100000
K = 20
D = 32
CHUNK = 32
E = CHUNK * K
NCHUNKS = N // CHUNK
NW = 32
GCALL = 640
NGC = E // GCALL
MAXT = (NCHUNKS + NW - 1) // NW
HALFT = MAXT // 2


def _body(nei_hbm, wei_hbm, emb_hbm, out_hbm,
          idx_v, wvec_v, rows_v, out_v,
          sem_i0, sem_i1, sem_g0, sem_g1, sem_w0, sem_w1, sem_o0, sem_o1):
    cid = lax.axis_index("c")
    sid = lax.axis_index("s")
    wid = sid * 2 + cid
    sem_i = (sem_i0, sem_i1)
    sem_g = (sem_g0, sem_g1)
    sem_w = (sem_w0, sem_w1)
    sem_o = (sem_o0, sem_o1)

    def chunk_of(t):
        return wid + t * NW

    def start_idx(t, b):
        @pl.when(chunk_of(t) < NCHUNKS)
        def _():
            pltpu.async_copy(nei_hbm.at[pl.ds(chunk_of(t) * E, E)],
                             idx_v.at[b], sem_i[b])

    def wait_idx(t, b):
        @pl.when(chunk_of(t) < NCHUNKS)
        def _():
            pltpu.make_async_copy(nei_hbm.at[pl.ds(chunk_of(t) * E, E)],
                                  idx_v.at[b], sem_i[b]).wait()

    def start_gather(t, b):
        @pl.when(chunk_of(t) < NCHUNKS)
        def _():
            for j in range(NGC):
                pltpu.async_copy(
                    emb_hbm.at[idx_v.at[b, pl.ds(j * GCALL, GCALL)]],
                    rows_v.at[b, pl.ds(j * GCALL, GCALL)],
                    sem_g[b],
                )
            pltpu.async_copy(
                wei_hbm.at[pl.ds(chunk_of(t) * CHUNK * 32, CHUNK * 32)],
                wvec_v.at[b], sem_w[b])

    def wait_gather(t, b):
        @pl.when(chunk_of(t) < NCHUNKS)
        def _():
            for j in range(NGC):
                pltpu.make_async_copy(
                    emb_hbm.at[idx_v.at[b, pl.ds(j * GCALL, GCALL)]],
                    rows_v.at[b, pl.ds(j * GCALL, GCALL)],
                    sem_g[b],
                ).wait()
            pltpu.make_async_copy(
                wei_hbm.at[pl.ds(chunk_of(t) * CHUNK * 32, CHUNK * 32)],
                wvec_v.at[b], sem_w[b])\
                .wait()

    def wait_out(t, b):
        c = chunk_of(t)

        @pl.when(jnp.logical_and(t >= 0, c < NCHUNKS))
        def _():
            pltpu.make_async_copy(
                out_v.at[b], out_hbm.at[pl.ds(chunk_of(t) * CHUNK, CHUNK)],
                sem_o[b]).wait()

    def compute(t, b):
        @pl.when(chunk_of(t) < NCHUNKS)
        def _():
            def row_body(r, carry2):
                e0 = r * K
                wa = wvec_v[b, pl.ds(r * 32, 16)]
                wb = wvec_v[b, pl.ds(r * 32 + 16, 16)]
                w0 = wa[0]
                acc0 = w0 * rows_v[b, e0, pl.ds(0, 16)]
                acc1 = w0 * rows_v[b, e0, pl.ds(16, 16)]
                for k in range(1, K):
                    w = wa[k] if k < 16 else wb[k - 16]
                    acc0 = acc0 + w * rows_v[b, e0 + k, pl.ds(0, 16)]
                    acc1 = acc1 + w * rows_v[b, e0 + k, pl.ds(16, 16)]
                out_v[b, r, pl.ds(0, 16)] = acc0
                out_v[b, r, pl.ds(16, 16)] = acc1
                return carry2

            lax.fori_loop(0, CHUNK, row_body, 0, unroll=4)
            pltpu.async_copy(
                out_v.at[b], out_hbm.at[pl.ds(chunk_of(t) * CHUNK, CHUNK)],
                sem_o[b])

    start_idx(0, 0)
    start_idx(1, 1)
    wait_idx(0, 0)
    start_gather(0, 0)

    def step(u, carry):
        t0 = 2 * u
        wait_idx(t0 + 1, 1)
        start_gather(t0 + 1, 1)
        wait_gather(t0, 0)
        start_idx(t0 + 2, 0)
        wait_out(t0 - 2, 0)
        compute(t0, 0)
        wait_idx(t0 + 2, 0)
        start_gather(t0 + 2, 0)
        wait_gather(t0 + 1, 1)
        start_idx(t0 + 3, 1)
        wait_out(t0 - 1, 1)
        compute(t0 + 1, 1)
        return carry

    lax.fori_loop(0, HALFT, step, 0)
    wait_out(MAXT - 2, 0)
    wait_out(MAXT - 1, 1)


@jax.jit
def _pprgo(nei_flat, wei_flat, emb_table):
    mesh = plsc.VectorSubcoreMesh(core_axis_name="c", subcore_axis_name="s")
    out = pl.kernel(
        _body,
        out_type=jax.ShapeDtypeStruct((N, D), jnp.float32),
        mesh=mesh,
        scratch_types=[
            pltpu.VMEM((2, E), jnp.int32),
            pltpu.VMEM((2, CHUNK * 32), jnp.float32),
            pltpu.VMEM((2, E, D), jnp.float32),
            pltpu.VMEM((2, CHUNK, D), jnp.float32),
        ] + [pltpu.SemaphoreType.DMA] * 8,
        compiler_params=pltpu.CompilerParams(use_tc_tiling_on_sc=False),
    )(nei_flat, wei_flat, emb_table)
    return out


def kernel(nei, wei, emb_table):
    nei_flat = nei.reshape(-1).astype(jnp.int32)
    wei_pad = jnp.pad(wei, ((0, 0), (0, 32 - K))).reshape(-1)
    out = _pprgo(nei_flat, wei_pad, emb_table)
    return out[:, None, :]
